# baseline (speedup 1.0000x reference)
import functools
import math

import jax
import jax.numpy as jnp
from jax import lax
from jax.experimental import pallas as pl
from jax.experimental.pallas import tpu as pltpu
from jax.experimental.pallas import tpu_sc as plsc

FEATS = 64
NPF = FEATS // 2
H = 64
W = 64
HW = H * W
B = 4
NUM_OFFSETS = 7
_TEMPERATURE = 10000.0
_SCALE = 2.0 * math.pi
_EPS = 1e-6

_NC = 2
_NS = 16
_L = 16
_NW = _NC * _NS
_CPW = FEATS // _NW


def _pe_table():
    c = jnp.arange(FEATS)[:, None]
    col = jnp.arange(HW)[None, :]
    y = col // W
    x = col - y * W
    is_y = c < NPF
    cm = jnp.where(is_y, c, c - NPF)
    k = cm // 2
    e = (jnp.where(is_y, y, x).astype(jnp.float32) + 1.0) * (
        _SCALE / (float(H) + _EPS)
    )
    inv_d = jnp.exp(k.astype(jnp.float32) * (-math.log(_TEMPERATURE) * 2.0 / NPF))
    arg = e * inv_d
    pe = jnp.where(cm % 2 == 0, jnp.sin(arg), jnp.cos(arg))
    return pe.reshape(FEATS * HW)


def _sc_body(offs_hbm, pos_hbm, pe_hbm, out_hbm, idx_v, off_v, pe_v, gsem, osem):
    wid = lax.axis_index("s") * _NC + lax.axis_index("c")
    c0 = wid * _CPW
    pltpu.sync_copy(pos_hbm, idx_v)
    gat = pltpu.async_copy(offs_hbm.at[idx_v], off_v, gsem)
    pltpu.sync_copy(pe_hbm.at[pl.ds(c0 * HW, _CPW * HW)], pe_v)
    gat.wait()
    for ch in range(_CPW):
        offv = off_v[0, pl.ds((c0 + ch) * _L, _L)]

        def body(j, carry, _ch=ch, _offv=offv):
            base = _ch * HW + j * _L
            pe_v[pl.ds(base, _L)] = pe_v[pl.ds(base, _L)] + _offv
            return carry

        lax.fori_loop(0, HW // _L, body, 0)
    copies = [
        pltpu.async_copy(
            pe_v, out_hbm.at[pl.ds((b * FEATS + c0) * HW, _CPW * HW)], osem
        )
        for b in range(B)
    ]
    for cp in copies:
        cp.wait()


def kernel(base_memposenc_offsets, imagelike_shape_bchw, position_offset):
    del imagelike_shape_bchw
    offs = base_memposenc_offsets.reshape(NUM_OFFSETS, FEATS)
    offs_exp = jnp.broadcast_to(
        offs[:, :, None], (NUM_OFFSETS, FEATS, _L)
    ).reshape(NUM_OFFSETS, FEATS * _L)
    pos1 = jnp.asarray(position_offset, jnp.int32).reshape(1)
    pe = _pe_table()
    run = functools.partial(
        pl.kernel,
        mesh=plsc.VectorSubcoreMesh(core_axis_name="c", subcore_axis_name="s"),
        out_type=jax.ShapeDtypeStruct((B * FEATS * HW,), jnp.float32),
        scratch_types=[
            pltpu.VMEM((1,), jnp.int32),
            pltpu.VMEM((1, FEATS * _L), jnp.float32),
            pltpu.VMEM((_CPW * HW,), jnp.float32),
            pltpu.SemaphoreType.DMA,
            pltpu.SemaphoreType.DMA,
        ],
    )(_sc_body)
    out = run(offs_exp, pos1, pe)
    return out.reshape(B, FEATS, H, W)


# speedup vs baseline: 2.2369x; 2.2369x over previous
import math

import jax
import jax.numpy as jnp
from jax.experimental import pallas as pl
from jax.experimental.pallas import tpu as pltpu

FEATS = 64
NPF = FEATS // 2
H = 64
W = 64
HW = H * W
B = 4
NUM_OFFSETS = 7
_TEMPERATURE = 10000.0
_SCALE = 2.0 * math.pi
_EPS = 1e-6


def _body(pos_ref, offt_ref, out_ref, sel_ref, sem):
    c = jax.lax.broadcasted_iota(jnp.int32, (FEATS, 1), 0)
    is_y = c < NPF
    cm = jnp.where(is_y, c, c - NPF)
    k = cm // 2
    inv_d = jnp.exp(k.astype(jnp.float32) * (-math.log(_TEMPERATURE) * 2.0 / NPF))
    phase = (cm % 2).astype(jnp.float32) * (0.5 * math.pi)
    pos = pos_ref[0, 0]
    lane = jax.lax.broadcasted_iota(jnp.int32, (FEATS, NUM_OFFSETS), 1)
    off = jnp.sum(
        jnp.where(lane == pos, offt_ref[...], 0.0), axis=1, keepdims=True
    )
    col = jax.lax.broadcasted_iota(jnp.int32, (FEATS, HW), 1)
    y = col // W
    x = col - y * W
    e = (jnp.where(is_y, y, x).astype(jnp.float32) + 1.0) * (
        _SCALE / (float(H) + _EPS)
    )
    sel_ref[...] = jnp.sin(e * inv_d + phase) + off
    copies = [
        pltpu.make_async_copy(sel_ref, out_ref.at[b], sem) for b in range(B)
    ]
    for cp in copies:
        cp.start()
    for cp in copies:
        cp.wait()


def kernel(base_memposenc_offsets, imagelike_shape_bchw, position_offset):
    del imagelike_shape_bchw
    offt = base_memposenc_offsets.reshape(NUM_OFFSETS, FEATS).T
    pos = jnp.asarray(position_offset, jnp.int32).reshape(1, 1)
    out = pl.pallas_call(
        _body,
        in_specs=[
            pl.BlockSpec(memory_space=pltpu.SMEM),
            pl.BlockSpec(memory_space=pltpu.VMEM),
        ],
        out_specs=pl.BlockSpec(memory_space=pl.ANY),
        out_shape=jax.ShapeDtypeStruct((B, FEATS, HW), jnp.float32),
        scratch_shapes=[
            pltpu.VMEM((FEATS, HW), jnp.float32),
            pltpu.SemaphoreType.DMA,
        ],
    )(pos, offt)
    return out.reshape(B, FEATS, H, W)


# speedup vs baseline: 2.3879x; 1.0675x over previous
import math

import jax
import jax.numpy as jnp
from jax.experimental import pallas as pl
from jax.experimental.pallas import tpu as pltpu

FEATS = 64
NPF = FEATS // 2
H = 64
W = 64
HW = H * W
B = 4
NUM_OFFSETS = 7
_TEMPERATURE = 10000.0
_SCALE = 2.0 * math.pi
_EPS = 1e-6
_NCHUNK = 8
_RC = FEATS // _NCHUNK


def _body(pos_ref, offt_ref, out_ref, sel_ref, sem):
    pos = pos_ref[0, 0]
    lane = jax.lax.broadcasted_iota(jnp.int32, (FEATS, NUM_OFFSETS), 1)
    off = jnp.sum(
        jnp.where(lane == pos, offt_ref[...], 0.0), axis=1, keepdims=True
    )

    col = jax.lax.broadcasted_iota(jnp.int32, (_RC, HW), 1)
    ey = ((col // W).astype(jnp.float32) + 1.0) * (_SCALE / (float(H) + _EPS))
    ex = ((col % W).astype(jnp.float32) + 1.0) * (_SCALE / (float(W) + _EPS))
    crow = jax.lax.broadcasted_iota(jnp.int32, (_RC, 1), 0)

    copies = []
    for j in range(_NCHUNK):
        c0 = j * _RC
        cm = crow + (c0 if c0 < NPF else c0 - NPF)
        k = cm // 2
        inv_d = jnp.exp(
            k.astype(jnp.float32) * (-math.log(_TEMPERATURE) * 2.0 / NPF)
        )
        phase = (cm % 2).astype(jnp.float32) * (0.5 * math.pi)
        e = ey if c0 < NPF else ex
        sel_ref[pl.ds(c0, _RC)] = jnp.sin(e * inv_d + phase) + off[c0 : c0 + _RC]
        for b in range(B):
            cp = pltpu.make_async_copy(
                sel_ref.at[pl.ds(c0, _RC)], out_ref.at[b, pl.ds(c0, _RC)], sem
            )
            cp.start()
            copies.append(cp)
    for cp in copies:
        cp.wait()


def kernel(base_memposenc_offsets, imagelike_shape_bchw, position_offset):
    del imagelike_shape_bchw
    offt = base_memposenc_offsets.reshape(NUM_OFFSETS, FEATS).T
    pos = jnp.asarray(position_offset, jnp.int32).reshape(1, 1)
    out = pl.pallas_call(
        _body,
        in_specs=[
            pl.BlockSpec(memory_space=pltpu.SMEM),
            pl.BlockSpec(memory_space=pltpu.VMEM),
        ],
        out_specs=pl.BlockSpec(memory_space=pl.ANY),
        out_shape=jax.ShapeDtypeStruct((B, FEATS, HW), jnp.float32),
        scratch_shapes=[
            pltpu.VMEM((FEATS, HW), jnp.float32),
            pltpu.SemaphoreType.DMA,
        ],
    )(pos, offt)
    return out.reshape(B, FEATS, H, W)


# speedup vs baseline: 2.4283x; 1.0169x over previous
import math

import jax
import jax.numpy as jnp
from jax.experimental import pallas as pl
from jax.experimental.pallas import tpu as pltpu

FEATS = 64
NPF = FEATS // 2
H = 64
W = 64
HW = H * W
B = 4
NUM_OFFSETS = 7
_TEMPERATURE = 10000.0
_SCALE = 2.0 * math.pi
_EPS = 1e-6
_NCHUNK = 8
_RC = FEATS // _NCHUNK


def _body(pos_ref, offs_ref, out_ref, sel_ref, sem):
    pos = pos_ref[0, 0]
    row = jax.lax.broadcasted_iota(jnp.int32, (NUM_OFFSETS, FEATS), 0)
    off_row = jnp.sum(
        jnp.where(row == pos, offs_ref[...], 0.0), axis=0, keepdims=True
    )
    ci = jax.lax.broadcasted_iota(jnp.int32, (FEATS, FEATS), 1)
    ri = jax.lax.broadcasted_iota(jnp.int32, (FEATS, FEATS), 0)
    off = jnp.sum(
        jnp.where(ci == ri, jnp.broadcast_to(off_row, (FEATS, FEATS)), 0.0),
        axis=1,
        keepdims=True,
    )

    col = jax.lax.broadcasted_iota(jnp.int32, (_RC, HW), 1)
    ey = ((col // W).astype(jnp.float32) + 1.0) * (_SCALE / (float(H) + _EPS))
    ex = ((col % W).astype(jnp.float32) + 1.0) * (_SCALE / (float(W) + _EPS))
    crow = jax.lax.broadcasted_iota(jnp.int32, (_RC, 1), 0)

    copies = []
    for j in range(_NCHUNK):
        c0 = j * _RC
        cm = crow + (c0 if c0 < NPF else c0 - NPF)
        k = cm // 2
        inv_d = jnp.exp(
            k.astype(jnp.float32) * (-math.log(_TEMPERATURE) * 2.0 / NPF)
        )
        phase = (cm % 2).astype(jnp.float32) * (0.5 * math.pi)
        e = ey if c0 < NPF else ex
        sel_ref[pl.ds(c0, _RC)] = jnp.sin(e * inv_d + phase) + off[c0 : c0 + _RC]
        for b in range(B):
            cp = pltpu.make_async_copy(
                sel_ref.at[pl.ds(c0, _RC)], out_ref.at[b, pl.ds(c0, _RC)], sem
            )
            cp.start()
            copies.append(cp)
    for cp in copies:
        cp.wait()


def kernel(base_memposenc_offsets, imagelike_shape_bchw, position_offset):
    del imagelike_shape_bchw
    offs = base_memposenc_offsets.reshape(NUM_OFFSETS, FEATS)
    pos = jnp.asarray(position_offset, jnp.int32).reshape(1, 1)
    out = pl.pallas_call(
        _body,
        in_specs=[
            pl.BlockSpec(memory_space=pltpu.SMEM),
            pl.BlockSpec(memory_space=pltpu.VMEM),
        ],
        out_specs=pl.BlockSpec(memory_space=pl.ANY),
        out_shape=jax.ShapeDtypeStruct((B, FEATS, HW), jnp.float32),
        scratch_shapes=[
            pltpu.VMEM((FEATS, HW), jnp.float32),
            pltpu.SemaphoreType.DMA,
        ],
    )(pos, offs)
    return out.reshape(B, FEATS, H, W)
